# initial kernel scaffold (unmeasured)
import jax
import jax.numpy as jnp
from jax import lax
from jax.experimental import pallas as pl
from jax.experimental.pallas import tpu as pltpu

N_DEV = 16


def kernel(x, w_mat, scale_x, scale_w):
    m_per, k = x.shape
    _, n_per = w_mat.shape

    x8 = x.astype(jnp.float8_e4m3fn)
    w_bf = w_mat.astype(jnp.bfloat16)
    scale = (scale_x * scale_w).astype(jnp.float32)

    def body(x_ref, w_ref, s_ref, out_ref, comm_ref, send_sems, recv_sems):
        my = lax.axis_index("i")
        right = lax.rem(my + 1, N_DEV)
        s = s_ref[0]

        def compute(chunk, origin):
            acc = lax.dot_general(
                chunk.astype(jnp.bfloat16), w_ref[...],
                (((1,), (0,)), ((), ())),
                preferred_element_type=jnp.float32,
            )
            out_ref[pl.ds(origin * m_per, m_per), :] = acc * s

        for h in range(N_DEV - 1):
            o_s = lax.rem(my - h + N_DEV, N_DEV)
            o_r = lax.rem(my - h - 1 + N_DEV, N_DEV)
            src = x_ref if h == 0 else comm_ref.at[o_s]
            send = pltpu.make_async_remote_copy(
                src_ref=src,
                dst_ref=comm_ref.at[o_s],
                send_sem=send_sems.at[o_s],
                recv_sem=recv_sems.at[o_s],
                device_id=(right,),
                device_id_type=pl.DeviceIdType.MESH,
            )
            send.start()
            compute(x_ref[...] if h == 0 else comm_ref[o_s], o_s)
            recv = pltpu.make_async_remote_copy(
                src_ref=comm_ref.at[o_r],
                dst_ref=comm_ref.at[o_r],
                send_sem=send_sems.at[o_s],
                recv_sem=recv_sems.at[o_r],
                device_id=(right,),
                device_id_type=pl.DeviceIdType.MESH,
            )
            send.wait_send()
            recv.wait_recv()

        o_last = lax.rem(my + 1, N_DEV)
        compute(comm_ref[o_last], o_last)

    return pl.pallas_call(
        body,
        out_shape=jax.ShapeDtypeStruct((N_DEV * m_per, n_per), jnp.float32),
        in_specs=[
            pl.BlockSpec(memory_space=pltpu.VMEM),
            pl.BlockSpec(memory_space=pltpu.VMEM),
            pl.BlockSpec(memory_space=pltpu.SMEM),
        ],
        out_specs=pl.BlockSpec(memory_space=pltpu.VMEM),
        scratch_shapes=[
            pltpu.VMEM((N_DEV, m_per, k), jnp.float8_e4m3fn),
            pltpu.SemaphoreType.DMA((N_DEV,)),
            pltpu.SemaphoreType.DMA((N_DEV,)),
        ],
        compiler_params=pltpu.CompilerParams(collective_id=0),
    )(x8, w_bf, scale)


# baseline (device time: 209784 ns/iter reference)
import jax
import jax.numpy as jnp
from jax import lax
from jax.experimental import pallas as pl
from jax.experimental.pallas import tpu as pltpu

N_DEV = 16


def kernel(x, w_mat, scale_x, scale_w):
    m_per, k = x.shape
    _, n_per = w_mat.shape

    x8 = x.astype(jnp.float8_e4m3fn)
    w_bf = w_mat.astype(jnp.bfloat16)
    scale = (scale_x * scale_w).astype(jnp.float32)

    def body(x_ref, w_ref, s_ref, out_ref, comm_ref, send_sems, recv_sems):
        my = lax.axis_index("i")
        right = lax.rem(my + 1, N_DEV)
        s = s_ref[0]

        def compute(chunk, origin):
            acc = lax.dot_general(
                chunk.astype(jnp.bfloat16), w_ref[...],
                (((1,), (0,)), ((), ())),
                preferred_element_type=jnp.float32,
            )
            out_ref[pl.ds(origin * m_per, m_per), :] = acc * s

        for h in range(N_DEV - 1):
            o_s = lax.rem(my - h + N_DEV, N_DEV)
            o_r = lax.rem(my - h - 1 + N_DEV, N_DEV)
            src = x_ref if h == 0 else comm_ref.at[o_s]
            send = pltpu.make_async_remote_copy(
                src_ref=src,
                dst_ref=comm_ref.at[o_s],
                send_sem=send_sems.at[o_s],
                recv_sem=recv_sems.at[o_s],
                device_id=(right,),
                device_id_type=pl.DeviceIdType.MESH,
            )
            send.start()
            compute(x_ref[...] if h == 0 else comm_ref[o_s], o_s)
            recv = pltpu.make_async_remote_copy(
                src_ref=comm_ref.at[o_r],
                dst_ref=comm_ref.at[o_r],
                send_sem=send_sems.at[o_s],
                recv_sem=recv_sems.at[o_r],
                device_id=(right,),
                device_id_type=pl.DeviceIdType.MESH,
            )
            send.wait_send()
            recv.wait_recv()

        o_last = lax.rem(my + 1, N_DEV)
        compute(comm_ref[o_last], o_last)

    return pl.pallas_call(
        body,
        out_shape=jax.ShapeDtypeStruct((N_DEV * m_per, n_per), jnp.float32),
        in_specs=[
            pl.BlockSpec(memory_space=pltpu.VMEM),
            pl.BlockSpec(memory_space=pltpu.VMEM),
            pl.BlockSpec(memory_space=pltpu.SMEM),
        ],
        out_specs=pl.BlockSpec(memory_space=pltpu.VMEM),
        scratch_shapes=[
            pltpu.VMEM((N_DEV, m_per, k), jnp.float8_e4m3fn),
            pltpu.SemaphoreType.DMA((N_DEV,)),
            pltpu.SemaphoreType.DMA((N_DEV,)),
        ],
    )(x8, w_bf, scale)


# device time: 129709 ns/iter; 1.6173x vs baseline; 1.6173x over previous
import jax
import jax.numpy as jnp
from jax import lax
from jax.experimental import pallas as pl
from jax.experimental.pallas import tpu as pltpu

N_DEV = 16
R_HOPS = 8
L_HOPS = 7


def kernel(x, w_mat, scale_x, scale_w):
    m_per, k = x.shape
    _, n_per = w_mat.shape

    x8 = x.astype(jnp.float8_e4m3fn)
    w_bf = w_mat.astype(jnp.bfloat16)
    scale = (scale_x * scale_w).astype(jnp.float32)

    def body(x_ref, w_ref, s_ref, out_ref, comm_ref,
             send_r, send_l, recv_r, recv_l):
        my = lax.axis_index("i")
        right = lax.rem(my + 1, N_DEV)
        left = lax.rem(my - 1 + N_DEV, N_DEV)
        s = s_ref[0]

        def compute(chunk, origin):
            acc = lax.dot_general(
                chunk.astype(jnp.bfloat16), w_ref[...],
                (((1,), (0,)), ((), ())),
                preferred_element_type=jnp.float32,
            )
            out_ref[pl.ds(origin * m_per, m_per), :] = acc * s

        def mk(src, origin, dev, ssems, rsems):
            return pltpu.make_async_remote_copy(
                src_ref=src,
                dst_ref=comm_ref.at[origin],
                send_sem=ssems.at[origin],
                recv_sem=rsems.at[origin],
                device_id=(dev,),
                device_id_type=pl.DeviceIdType.MESH,
            )

        sends = []
        for h in range(R_HOPS):
            o_sr = lax.rem(my - h + N_DEV, N_DEV)
            rs = mk(x_ref if h == 0 else comm_ref.at[o_sr], o_sr,
                    right, send_r, recv_r)
            rs.start()
            sends.append(rs)
            if h < L_HOPS:
                o_sl = lax.rem(my + h, N_DEV)
                ls = mk(x_ref if h == 0 else comm_ref.at[o_sl], o_sl,
                        left, send_l, recv_l)
                ls.start()
                sends.append(ls)

            if h == 0:
                compute(x_ref[...], my)
            else:
                compute(comm_ref[o_sr], o_sr)
                compute(comm_ref[o_sl], o_sl)

            o_rr = lax.rem(my - 1 - h + N_DEV, N_DEV)
            mk(comm_ref.at[o_rr], o_rr, right, send_r, recv_r).wait_recv()
            if h < L_HOPS:
                o_rl = lax.rem(my + 1 + h, N_DEV)
                mk(comm_ref.at[o_rl], o_rl, left, send_l, recv_l).wait_recv()

        o8 = lax.rem(my + 8, N_DEV)
        compute(comm_ref[o8], o8)
        o7 = lax.rem(my + 7, N_DEV)
        compute(comm_ref[o7], o7)

        for rs in sends:
            rs.wait_send()

    return pl.pallas_call(
        body,
        out_shape=jax.ShapeDtypeStruct((N_DEV * m_per, n_per), jnp.float32),
        in_specs=[
            pl.BlockSpec(memory_space=pltpu.VMEM),
            pl.BlockSpec(memory_space=pltpu.VMEM),
            pl.BlockSpec(memory_space=pltpu.SMEM),
        ],
        out_specs=pl.BlockSpec(memory_space=pltpu.VMEM),
        scratch_shapes=[
            pltpu.VMEM((N_DEV, m_per, k), jnp.float8_e4m3fn),
            pltpu.SemaphoreType.DMA((N_DEV,)),
            pltpu.SemaphoreType.DMA((N_DEV,)),
            pltpu.SemaphoreType.DMA((N_DEV,)),
            pltpu.SemaphoreType.DMA((N_DEV,)),
        ],
    )(x8, w_bf, scale)


# device time: 103323 ns/iter; 2.0304x vs baseline; 1.2554x over previous
import jax
import jax.numpy as jnp
from jax import lax
from jax.experimental import pallas as pl
from jax.experimental.pallas import tpu as pltpu

N_DEV = 16


def kernel(x, w_mat, scale_x, scale_w):
    m_per, k = x.shape
    _, n_per = w_mat.shape
    half = m_per // 2

    x8 = x.astype(jnp.float8_e4m3fn)
    w_bf = w_mat.astype(jnp.bfloat16)
    scale = (scale_x * scale_w).astype(jnp.float32)

    def body(x_ref, w_ref, s_ref, out_ref, comm_ref,
             send_ra, send_rb, send_la, send_lb,
             recv_ra, recv_rb, recv_la, recv_lb):
        my = lax.axis_index("i")
        right = lax.rem(my + 1, N_DEV)
        left = lax.rem(my - 1 + N_DEV, N_DEV)
        s = s_ref[0]

        def compute(chunk, origin):
            acc = lax.dot_general(
                chunk.astype(jnp.bfloat16), w_ref[...],
                (((1,), (0,)), ((), ())),
                preferred_element_type=jnp.float32,
            )
            out_ref[pl.ds(origin * m_per, m_per), :] = acc * s

        def mk(src, o, lo, dev, ssems, rsems):
            return pltpu.make_async_remote_copy(
                src_ref=src,
                dst_ref=comm_ref.at[o, pl.ds(lo, half)],
                send_sem=ssems.at[o],
                recv_sem=rsems.at[o],
                device_id=(dev,),
                device_id_type=pl.DeviceIdType.MESH,
            )

        def fwd(o, lo, dev, ssems, rsems):
            return mk(comm_ref.at[o, pl.ds(lo, half)], o, lo, dev,
                      ssems, rsems)

        sends = []

        def start(d):
            d.start()
            sends.append(d)

        xa = x_ref.at[pl.ds(0, half)]
        xb = x_ref.at[pl.ds(half, half)]
        start(mk(xa, my, 0, right, send_ra, recv_ra))
        start(mk(xb, my, half, left, send_lb, recv_lb))
        start(mk(xb, my, half, right, send_rb, recv_rb))
        start(mk(xa, my, 0, left, send_la, recv_la))
        compute(x_ref[...], my)

        for h in range(7):
            o_r = lax.rem(my - 1 - h + N_DEV, N_DEV)
            o_l = lax.rem(my + 1 + h, N_DEV)
            ra = fwd(o_r, 0, right, send_ra, recv_ra)
            ra.wait_recv()
            ra.start()
            sends.append(ra)
            lb = fwd(o_l, half, left, send_lb, recv_lb)
            lb.wait_recv()
            lb.start()
            sends.append(lb)
            rb = fwd(o_r, half, right, send_rb, recv_rb)
            rb.wait_recv()
            if h < 6:
                rb.start()
                sends.append(rb)
            la = fwd(o_l, 0, left, send_la, recv_la)
            la.wait_recv()
            if h < 6:
                la.start()
                sends.append(la)
            compute(comm_ref[o_r], o_r)
            compute(comm_ref[o_l], o_l)

        o8 = lax.rem(my + 8, N_DEV)
        fwd(o8, 0, right, send_ra, recv_ra).wait_recv()
        fwd(o8, half, left, send_lb, recv_lb).wait_recv()
        compute(comm_ref[o8], o8)

        for d in sends:
            d.wait_send()

    return pl.pallas_call(
        body,
        out_shape=jax.ShapeDtypeStruct((N_DEV * m_per, n_per), jnp.float32),
        in_specs=[
            pl.BlockSpec(memory_space=pltpu.VMEM),
            pl.BlockSpec(memory_space=pltpu.VMEM),
            pl.BlockSpec(memory_space=pltpu.SMEM),
        ],
        out_specs=pl.BlockSpec(memory_space=pltpu.VMEM),
        scratch_shapes=[
            pltpu.VMEM((N_DEV, m_per, k), jnp.float8_e4m3fn),
            pltpu.SemaphoreType.DMA((N_DEV,)),
            pltpu.SemaphoreType.DMA((N_DEV,)),
            pltpu.SemaphoreType.DMA((N_DEV,)),
            pltpu.SemaphoreType.DMA((N_DEV,)),
            pltpu.SemaphoreType.DMA((N_DEV,)),
            pltpu.SemaphoreType.DMA((N_DEV,)),
            pltpu.SemaphoreType.DMA((N_DEV,)),
            pltpu.SemaphoreType.DMA((N_DEV,)),
        ],
    )(x8, w_bf, scale)


# device time: 98326 ns/iter; 2.1336x vs baseline; 1.0508x over previous
import jax
import jax.numpy as jnp
from jax import lax
from jax.experimental import pallas as pl
from jax.experimental.pallas import tpu as pltpu

N_DEV = 16


def kernel(x, w_mat, scale_x, scale_w):
    m_per, k = x.shape
    _, n_per = w_mat.shape
    half = m_per // 2

    x8 = x.astype(jnp.float8_e4m3fn)
    w_bf = w_mat.astype(jnp.bfloat16)
    scale = (scale_x * scale_w).astype(jnp.float32)

    def body(x_ref, w_ref, s_ref, out_ref, comm_ref,
             send_ra, send_rb, send_la, send_lb,
             recv_ra, recv_rb, recv_la, recv_lb):
        my = lax.axis_index("i")
        right = lax.rem(my + 1, N_DEV)
        left = lax.rem(my - 1 + N_DEV, N_DEV)
        s = s_ref[0]

        barrier_sem = pltpu.get_barrier_semaphore()
        for nbr in (left, right):
            pl.semaphore_signal(barrier_sem, inc=1, device_id=(nbr,),
                                device_id_type=pl.DeviceIdType.MESH)
        pl.semaphore_wait(barrier_sem, 2)

        def compute(chunk, origin, lo=0):
            acc = lax.dot_general(
                chunk.astype(jnp.bfloat16), w_ref[...],
                (((1,), (0,)), ((), ())),
                preferred_element_type=jnp.float32,
            )
            rows = chunk.shape[0]
            out_ref[pl.ds(origin * m_per + lo, rows), :] = acc * s

        def mk(src, o, lo, dev, ssems, rsems):
            return pltpu.make_async_remote_copy(
                src_ref=src,
                dst_ref=comm_ref.at[o, pl.ds(lo, half)],
                send_sem=ssems.at[o],
                recv_sem=rsems.at[o],
                device_id=(dev,),
                device_id_type=pl.DeviceIdType.MESH,
            )

        def fwd(o, lo, dev, ssems, rsems):
            return mk(comm_ref.at[o, pl.ds(lo, half)], o, lo, dev,
                      ssems, rsems)

        sends = []

        def start(d):
            d.start()
            sends.append(d)

        xa = x_ref.at[pl.ds(0, half)]
        xb = x_ref.at[pl.ds(half, half)]
        start(mk(xa, my, 0, right, send_ra, recv_ra))
        start(mk(xb, my, half, left, send_lb, recv_lb))
        start(mk(xb, my, half, right, send_rb, recv_rb))
        start(mk(xa, my, 0, left, send_la, recv_la))
        compute(x_ref[...], my)

        for h in range(7):
            o_r = lax.rem(my - 1 - h + N_DEV, N_DEV)
            o_l = lax.rem(my + 1 + h, N_DEV)
            ra = fwd(o_r, 0, right, send_ra, recv_ra)
            ra.wait_recv()
            ra.start()
            sends.append(ra)
            lb = fwd(o_l, half, left, send_lb, recv_lb)
            lb.wait_recv()
            lb.start()
            sends.append(lb)
            rb = fwd(o_r, half, right, send_rb, recv_rb)
            rb.wait_recv()
            if h < 6:
                rb.start()
                sends.append(rb)
            la = fwd(o_l, 0, left, send_la, recv_la)
            la.wait_recv()
            if h < 6:
                la.start()
                sends.append(la)
            compute(comm_ref[o_r], o_r)
            compute(comm_ref[o_l], o_l)

        o8 = lax.rem(my + 8, N_DEV)
        fwd(o8, 0, right, send_ra, recv_ra).wait_recv()
        compute(comm_ref[o8, pl.ds(0, half)], o8, 0)
        fwd(o8, half, left, send_lb, recv_lb).wait_recv()
        compute(comm_ref[o8, pl.ds(half, half)], o8, half)

        for d in sends:
            d.wait_send()

    return pl.pallas_call(
        body,
        out_shape=jax.ShapeDtypeStruct((N_DEV * m_per, n_per), jnp.float32),
        in_specs=[
            pl.BlockSpec(memory_space=pltpu.VMEM),
            pl.BlockSpec(memory_space=pltpu.VMEM),
            pl.BlockSpec(memory_space=pltpu.SMEM),
        ],
        out_specs=pl.BlockSpec(memory_space=pltpu.VMEM),
        scratch_shapes=[
            pltpu.VMEM((N_DEV, m_per, k), jnp.float8_e4m3fn),
            pltpu.SemaphoreType.DMA((N_DEV,)),
            pltpu.SemaphoreType.DMA((N_DEV,)),
            pltpu.SemaphoreType.DMA((N_DEV,)),
            pltpu.SemaphoreType.DMA((N_DEV,)),
            pltpu.SemaphoreType.DMA((N_DEV,)),
            pltpu.SemaphoreType.DMA((N_DEV,)),
            pltpu.SemaphoreType.DMA((N_DEV,)),
            pltpu.SemaphoreType.DMA((N_DEV,)),
        ],
        compiler_params=pltpu.CompilerParams(collective_id=0),
    )(x8, w_bf, scale)
